# baseline (speedup 1.0000x reference)
import jax
import jax.numpy as jnp
from jax import lax
from jax.experimental import pallas as pl
from jax.experimental.pallas import tpu as pltpu
from jax.experimental.pallas import tpu_sc as plsc

_RES = 100000
_EPS = 1e-06
_NQ = 16777216
_NC = 2
_NS = 16
_NW = _NC * _NS
_L = 16
_QPW = _NQ // _NW
_CHUNK = 4096
_NCHUNK = _QPW // _CHUNK
_NPAIR = _NCHUNK // 2

_STEP = (1.0 - 2.0 * _EPS) / (_RES - 1)
_INV_STEP = 1.0 / _STEP
_NEG_OFF = -_EPS * _INV_STEP
_TABN = _RES + _L


def _body(mq_hbm, mv_hbm, kv_hbm, out_hbm, tab,
          ib0, ib1, ob0, ob1, si0, si1, so0, so1):
    wid = lax.axis_index("s") * _NC + lax.axis_index("c")
    base = wid * _QPW
    pltpu.sync_copy(kv_hbm, tab.at[pl.ds(0, _RES)])
    k_last = plsc.load_gather(tab, [jnp.full((_L,), _RES - 1, jnp.int32)])
    tab[pl.ds(_RES, _L)] = k_last
    ibufs, obufs = (ib0, ib1), (ob0, ob1)
    sis, sos = (si0, si1), (so0, so1)

    def in_copy(g, b):
        return pltpu.make_async_copy(
            mq_hbm.at[pl.ds(base + g * _CHUNK, _CHUNK)], ibufs[b], sis[b])

    def out_copy(g, b):
        return pltpu.make_async_copy(
            obufs[b], out_hbm.at[pl.ds(base + g * _CHUNK, _CHUNK)], sos[b])

    def compute(b):
        ib, ob = ibufs[b], obufs[b]

        @plsc.parallel_loop(0, _CHUNK, _L, unroll=32)
        def _vec_body(o):
            q = ib[pl.ds(o, _L)]
            t = q * _INV_STEP + _NEG_OFF
            i = t.astype(jnp.int32)
            frac = t - i.astype(jnp.float32)
            y0 = plsc.load_gather(tab, [i])
            y1 = plsc.load_gather(tab, [i + 1])
            ob[pl.ds(o, _L)] = y0 + (y1 - y0) * frac

    in_copy(0, 0).start()

    def pair(p, carry):
        for b in range(2):
            g = 2 * p + b
            nb = 1 - b
            if b == 0:
                in_copy(g + 1, nb).start()
            else:
                @pl.when(p < _NPAIR - 1)
                def _():
                    in_copy(g + 1, nb).start()
            in_copy(g, b).wait()

            @pl.when(p > 0)
            def _():
                out_copy(g - 2, b).wait()

            compute(b)
            out_copy(g, b).start()
        return carry

    lax.fori_loop(0, _NPAIR, pair, 0)
    out_copy(_NCHUNK - 2, 0).wait()
    out_copy(_NCHUNK - 1, 1).wait()


def kernel(m_query, m_vals, K_vals):
    mesh = plsc.VectorSubcoreMesh(core_axis_name="c", subcore_axis_name="s")
    f = pl.kernel(
        _body,
        mesh=mesh,
        out_type=jax.ShapeDtypeStruct((_NQ,), jnp.float32),
        scratch_types=[
            pltpu.VMEM((_TABN,), jnp.float32),
            pltpu.VMEM((_CHUNK,), jnp.float32),
            pltpu.VMEM((_CHUNK,), jnp.float32),
            pltpu.VMEM((_CHUNK,), jnp.float32),
            pltpu.VMEM((_CHUNK,), jnp.float32),
            pltpu.SemaphoreType.DMA,
            pltpu.SemaphoreType.DMA,
            pltpu.SemaphoreType.DMA,
            pltpu.SemaphoreType.DMA,
        ],
        compiler_params=pltpu.CompilerParams(needs_layout_passes=False),
    )
    return f(m_query, m_vals, K_vals)


# speedup vs baseline: 2.1431x; 2.1431x over previous
import jax
import jax.numpy as jnp
from jax import lax
from jax.experimental import pallas as pl
from jax.experimental.pallas import tpu as pltpu
from jax.experimental.pallas import tpu_sc as plsc

_RES = 100000
_EPS = 1e-06
_NQ = 16777216
_NC = 2
_NS = 16
_NW = _NC * _NS
_L = 16
_QPW = _NQ // _NW
_CHUNK = 4096
_NCHUNK = _QPW // _CHUNK
_NPAIR = _NCHUNK // 2

_STEP = (1.0 - 2.0 * _EPS) / (_RES - 1)
_INV_STEP = 1.0 / _STEP
_NEG_OFF = -_EPS * _INV_STEP
_TABN = _RES + _L


def _body(mq_hbm, mv_hbm, kv_hbm, out_hbm, tab,
          ib0, ib1, ob0, ob1, si0, si1, so0, so1):
    wid = lax.axis_index("s") * _NC + lax.axis_index("c")
    base = wid * _QPW
    pltpu.sync_copy(kv_hbm, tab.at[pl.ds(0, _RES)])
    k_last = plsc.load_gather(tab, [jnp.full((_L,), _RES - 1, jnp.int32)])
    tab[pl.ds(_RES, _L)] = k_last
    ibufs, obufs = (ib0, ib1), (ob0, ob1)
    sis, sos = (si0, si1), (so0, so1)

    def in_copy(g, b):
        return pltpu.make_async_copy(
            mq_hbm.at[pl.ds(base + g * _CHUNK, _CHUNK)], ibufs[b], sis[b])

    def out_copy(g, b):
        return pltpu.make_async_copy(
            obufs[b], out_hbm.at[pl.ds(base + g * _CHUNK, _CHUNK)], sos[b])

    def compute(b):
        ib, ob = ibufs[b], obufs[b]

        @plsc.parallel_loop(0, _CHUNK, _L, unroll=16)
        def _vec_body(o):
            q = ib[pl.ds(o, _L)]
            t = q * _INV_STEP + _NEG_OFF
            i = t.astype(jnp.int32)
            frac = t - i.astype(jnp.float32)
            y0 = plsc.load_gather(tab, [i])
            y1 = plsc.load_gather(tab, [i + 1])
            ob[pl.ds(o, _L)] = y0 + (y1 - y0) * frac

    in_copy(0, 0).start()

    def pair(p, carry):
        for b in range(2):
            g = 2 * p + b
            nb = 1 - b
            if b == 0:
                in_copy(g + 1, nb).start()
            else:
                @pl.when(p < _NPAIR - 1)
                def _():
                    in_copy(g + 1, nb).start()
            in_copy(g, b).wait()

            @pl.when(p > 0)
            def _():
                out_copy(g - 2, b).wait()

            compute(b)
            out_copy(g, b).start()
        return carry

    lax.fori_loop(0, _NPAIR, pair, 0)
    out_copy(_NCHUNK - 2, 0).wait()
    out_copy(_NCHUNK - 1, 1).wait()


def kernel(m_query, m_vals, K_vals):
    mesh = plsc.VectorSubcoreMesh(core_axis_name="c", subcore_axis_name="s")
    f = pl.kernel(
        _body,
        mesh=mesh,
        out_type=jax.ShapeDtypeStruct((_NQ,), jnp.float32),
        scratch_types=[
            pltpu.VMEM((_TABN,), jnp.float32),
            pltpu.VMEM((_CHUNK,), jnp.float32),
            pltpu.VMEM((_CHUNK,), jnp.float32),
            pltpu.VMEM((_CHUNK,), jnp.float32),
            pltpu.VMEM((_CHUNK,), jnp.float32),
            pltpu.SemaphoreType.DMA,
            pltpu.SemaphoreType.DMA,
            pltpu.SemaphoreType.DMA,
            pltpu.SemaphoreType.DMA,
        ],
        compiler_params=pltpu.CompilerParams(needs_layout_passes=False),
    )
    return f(m_query, m_vals, K_vals)


# speedup vs baseline: 2.3066x; 1.0763x over previous
import jax
import jax.numpy as jnp
from jax import lax
from jax.experimental import pallas as pl
from jax.experimental.pallas import tpu as pltpu
from jax.experimental.pallas import tpu_sc as plsc

_RES = 100000
_EPS = 1e-06
_NQ = 16777216
_NC = 2
_NS = 16
_NW = _NC * _NS
_L = 16
_QPW = _NQ // _NW
_CHUNK = 4096
_NCHUNK = _QPW // _CHUNK
_NPAIR = _NCHUNK // 2

_STEP = (1.0 - 2.0 * _EPS) / (_RES - 1)
_INV_STEP = 1.0 / _STEP
_NEG_OFF = -_EPS * _INV_STEP
_TMAX = float(_RES - 1) - 0.0078125


def _body(mq_hbm, mv_hbm, kv_hbm, out_hbm, tab,
          ib0, ib1, ob0, ob1, si0, si1, so0, so1):
    wid = lax.axis_index("s") * _NC + lax.axis_index("c")
    base = wid * _QPW
    pltpu.sync_copy(kv_hbm, tab)
    ibufs, obufs = (ib0, ib1), (ob0, ob1)
    sis, sos = (si0, si1), (so0, so1)

    def in_copy(g, b):
        return pltpu.make_async_copy(
            mq_hbm.at[pl.ds(base + g * _CHUNK, _CHUNK)], ibufs[b], sis[b])

    def out_copy(g, b):
        return pltpu.make_async_copy(
            obufs[b], out_hbm.at[pl.ds(base + g * _CHUNK, _CHUNK)], sos[b])

    def compute(b):
        ib, ob = ibufs[b], obufs[b]

        @plsc.parallel_loop(0, _CHUNK, _L, unroll=8)
        def _vec_body(o):
            q = ib[pl.ds(o, _L)]
            t = jnp.minimum(q * _INV_STEP + _NEG_OFF, _TMAX)
            i = t.astype(jnp.int32)
            frac = t - i.astype(jnp.float32)
            y0 = plsc.load_gather(tab, [i])
            y1 = plsc.load_gather(tab, [i + 1])
            ob[pl.ds(o, _L)] = y0 + (y1 - y0) * frac

    in_copy(0, 0).start()

    def pair(p, carry):
        for b in range(2):
            g = 2 * p + b
            nb = 1 - b
            if b == 0:
                in_copy(g + 1, nb).start()
            else:
                @pl.when(p < _NPAIR - 1)
                def _():
                    in_copy(g + 1, nb).start()
            in_copy(g, b).wait()

            @pl.when(p > 0)
            def _():
                out_copy(g - 2, b).wait()

            compute(b)
            out_copy(g, b).start()
        return carry

    lax.fori_loop(0, _NPAIR, pair, 0)
    out_copy(_NCHUNK - 2, 0).wait()
    out_copy(_NCHUNK - 1, 1).wait()


def kernel(m_query, m_vals, K_vals):
    mesh = plsc.VectorSubcoreMesh(core_axis_name="c", subcore_axis_name="s")
    f = pl.kernel(
        _body,
        mesh=mesh,
        out_type=jax.ShapeDtypeStruct((_NQ,), jnp.float32),
        scratch_types=[
            pltpu.VMEM((_RES,), jnp.float32),
            pltpu.VMEM((_CHUNK,), jnp.float32),
            pltpu.VMEM((_CHUNK,), jnp.float32),
            pltpu.VMEM((_CHUNK,), jnp.float32),
            pltpu.VMEM((_CHUNK,), jnp.float32),
            pltpu.SemaphoreType.DMA,
            pltpu.SemaphoreType.DMA,
            pltpu.SemaphoreType.DMA,
            pltpu.SemaphoreType.DMA,
        ],
        compiler_params=pltpu.CompilerParams(needs_layout_passes=False),
    )
    return f(m_query, m_vals, K_vals)


# speedup vs baseline: 2.6391x; 1.1441x over previous
import jax
import jax.numpy as jnp
from jax import lax
from jax.experimental import pallas as pl
from jax.experimental.pallas import tpu as pltpu
from jax.experimental.pallas import tpu_sc as plsc

_RES = 100000
_EPS = 1e-06
_NQ = 16777216
_NC = 2
_NS = 16
_NW = _NC * _NS
_L = 16
_QPW = _NQ // _NW
_CHUNK = 8192
_NCHUNK = _QPW // _CHUNK
_NTRIPLE = _NCHUNK // 3

_STEP = (1.0 - 2.0 * _EPS) / (_RES - 1)
_INV_STEP = 1.0 / _STEP
_NEG_OFF = -_EPS * _INV_STEP
_TMAX = float(_RES - 1) - 0.0078125


def _body(mq_hbm, mv_hbm, kv_hbm, out_hbm, tab,
          b0, b1, b2, si0, si1, si2, so0, so1, so2):
    wid = lax.axis_index("s") * _NC + lax.axis_index("c")
    base = wid * _QPW
    pltpu.sync_copy(kv_hbm, tab)
    bufs = (b0, b1, b2)
    sis, sos = (si0, si1, si2), (so0, so1, so2)

    def in_copy(g, r):
        return pltpu.make_async_copy(
            mq_hbm.at[pl.ds(base + g * _CHUNK, _CHUNK)], bufs[r], sis[r])

    def out_copy(g, r):
        return pltpu.make_async_copy(
            bufs[r], out_hbm.at[pl.ds(base + g * _CHUNK, _CHUNK)], sos[r])

    def compute(r):
        buf = bufs[r]

        @plsc.parallel_loop(0, _CHUNK, _L, unroll=8)
        def _vec_body(o):
            q = buf[pl.ds(o, _L)]
            t = jnp.minimum(q * _INV_STEP + _NEG_OFF, _TMAX)
            i = t.astype(jnp.int32)
            frac = t - i.astype(jnp.float32)
            y0 = plsc.load_gather(tab, [i])
            y1 = plsc.load_gather(tab, [i + 1])
            buf[pl.ds(o, _L)] = y0 + (y1 - y0) * frac

    in_copy(0, 0).start()

    def triple(p, carry):
        for b in range(3):
            g = 3 * p + b
            r = b
            rn = (b + 1) % 3
            if b == 2:
                out_copy(g - 2, rn).wait()
            else:
                @pl.when(p > 0)
                def _():
                    out_copy(g - 2, rn).wait()
            in_copy(g + 1, rn).start()
            in_copy(g, r).wait()
            compute(r)
            out_copy(g, r).start()
        return carry

    lax.fori_loop(0, _NTRIPLE, triple, 0)
    g_last = _NCHUNK - 1
    in_copy(g_last, 0).wait()
    compute(0)
    out_copy(g_last, 0).start()
    out_copy(g_last - 2, 1).wait()
    out_copy(g_last - 1, 2).wait()
    out_copy(g_last, 0).wait()


def kernel(m_query, m_vals, K_vals):
    mesh = plsc.VectorSubcoreMesh(core_axis_name="c", subcore_axis_name="s")
    f = pl.kernel(
        _body,
        mesh=mesh,
        out_type=jax.ShapeDtypeStruct((_NQ,), jnp.float32),
        scratch_types=[
            pltpu.VMEM((_RES,), jnp.float32),
            pltpu.VMEM((_CHUNK,), jnp.float32),
            pltpu.VMEM((_CHUNK,), jnp.float32),
            pltpu.VMEM((_CHUNK,), jnp.float32),
            pltpu.SemaphoreType.DMA,
            pltpu.SemaphoreType.DMA,
            pltpu.SemaphoreType.DMA,
            pltpu.SemaphoreType.DMA,
            pltpu.SemaphoreType.DMA,
            pltpu.SemaphoreType.DMA,
        ],
        compiler_params=pltpu.CompilerParams(needs_layout_passes=False),
    )
    return f(m_query, m_vals, K_vals)


# speedup vs baseline: 2.8070x; 1.0636x over previous
import jax
import jax.numpy as jnp
from jax import lax
from jax.experimental import pallas as pl
from jax.experimental.pallas import tpu as pltpu
from jax.experimental.pallas import tpu_sc as plsc

_RES = 100000
_EPS = 1e-06
_NQ = 16777216
_NC = 2
_NS = 16
_NW = _NC * _NS
_L = 16
_QPW = _NQ // _NW
_CHUNK = 8192
_NCHUNK = _QPW // _CHUNK
_NTRIPLE = _NCHUNK // 3

_STEP = (1.0 - 2.0 * _EPS) / (_RES - 1)
_INV_STEP = 1.0 / _STEP
_NEG_OFF = -_EPS * _INV_STEP
_TMAX = float(_RES - 1) - 0.0078125
_TABN = _RES + _L


def _body(mq_hbm, mv_hbm, kv_hbm, out_hbm, tab,
          b0, b1, b2, si0, si1, si2, so0, so1, so2):
    wid = lax.axis_index("s") * _NC + lax.axis_index("c")
    base = wid * _QPW
    pltpu.sync_copy(kv_hbm, tab.at[pl.ds(0, _RES)])
    tab[pl.ds(_RES, _L)] = plsc.load_gather(
        tab, [jnp.full((_L,), _RES - 1, jnp.int32)])
    bufs = (b0, b1, b2)
    sis, sos = (si0, si1, si2), (so0, so1, so2)

    def in_copy(g, r):
        return pltpu.make_async_copy(
            mq_hbm.at[pl.ds(base + g * _CHUNK, _CHUNK)], bufs[r], sis[r])

    def out_copy(g, r):
        return pltpu.make_async_copy(
            bufs[r], out_hbm.at[pl.ds(base + g * _CHUNK, _CHUNK)], sos[r])

    def compute(r):
        buf = bufs[r]

        @plsc.parallel_loop(0, _CHUNK, _L, unroll=8)
        def _vec_body(o):
            q = buf[pl.ds(o, _L)]
            t = q * _INV_STEP + _NEG_OFF
            i = t.astype(jnp.int32)
            frac = t - i.astype(jnp.float32)
            y0 = plsc.load_gather(tab, [i])
            y1 = plsc.load_gather(tab, [i + 1])
            buf[pl.ds(o, _L)] = y0 + (y1 - y0) * frac

    in_copy(0, 0).start()

    def triple(p, carry):
        for b in range(3):
            g = 3 * p + b
            r = b
            rn = (b + 1) % 3
            if b == 2:
                out_copy(g - 2, rn).wait()
            else:
                @pl.when(p > 0)
                def _():
                    out_copy(g - 2, rn).wait()
            in_copy(g + 1, rn).start()
            in_copy(g, r).wait()
            compute(r)
            out_copy(g, r).start()
        return carry

    lax.fori_loop(0, _NTRIPLE, triple, 0)
    g_last = _NCHUNK - 1
    in_copy(g_last, 0).wait()
    compute(0)
    out_copy(g_last, 0).start()
    out_copy(g_last - 2, 1).wait()
    out_copy(g_last - 1, 2).wait()
    out_copy(g_last, 0).wait()


def kernel(m_query, m_vals, K_vals):
    mesh = plsc.VectorSubcoreMesh(core_axis_name="c", subcore_axis_name="s")
    f = pl.kernel(
        _body,
        mesh=mesh,
        out_type=jax.ShapeDtypeStruct((_NQ,), jnp.float32),
        scratch_types=[
            pltpu.VMEM((_TABN,), jnp.float32),
            pltpu.VMEM((_CHUNK,), jnp.float32),
            pltpu.VMEM((_CHUNK,), jnp.float32),
            pltpu.VMEM((_CHUNK,), jnp.float32),
            pltpu.SemaphoreType.DMA,
            pltpu.SemaphoreType.DMA,
            pltpu.SemaphoreType.DMA,
            pltpu.SemaphoreType.DMA,
            pltpu.SemaphoreType.DMA,
            pltpu.SemaphoreType.DMA,
        ],
        compiler_params=pltpu.CompilerParams(needs_layout_passes=False),
    )
    return f(m_query, m_vals, K_vals)
